# baseline (speedup 1.0000x reference)
import functools

import jax
import jax.numpy as jnp
from jax import lax
from jax.experimental import pallas as pl
from jax.experimental.pallas import tpu as pltpu
from jax.experimental.pallas import tpu_sc as plsc

VOCAB = 100000
HID = 128
MAXPOS = 512
VDIM = 2048
B = 1024
S = 200
NB = 36
EPS = 1e-12

NC = 2
NS = 16
NW = NC * NS
TOTAL = B * S
PER_W = TOTAL // NW
CHUNK = 128
NCHUNK = PER_W // CHUNK
NPAIR = NCHUNK // 2


def _word_gather_sc(tid, word_emb):
    mesh = plsc.VectorSubcoreMesh(core_axis_name="c", subcore_axis_name="s")

    @functools.partial(
        pl.kernel,
        out_type=jax.ShapeDtypeStruct((TOTAL, HID), jnp.float32),
        mesh=mesh,
        scratch_types=[
            pltpu.VMEM((2, CHUNK), jnp.int32),
            pltpu.VMEM((2, CHUNK, HID), jnp.float32),
            pltpu.SemaphoreType.DMA,
            pltpu.SemaphoreType.DMA,
            pltpu.SemaphoreType.DMA,
            pltpu.SemaphoreType.DMA,
        ],
    )
    def gather_kernel(tid_h, wtab_h, out_h, tid_v, wbuf, gs0, gs1, os0, os1):
        wid = lax.axis_index("s") * NC + lax.axis_index("c")
        base_w = wid * PER_W
        gsems = (gs0, gs1)
        osems = (os0, os1)

        def gather_desc(b):
            return pltpu.make_async_copy(wtab_h.at[tid_v.at[b]], wbuf.at[b],
                                         gsems[b])

        def fire(c, b):
            base = base_w + c * CHUNK
            pltpu.sync_copy(tid_h.at[pl.ds(base, CHUNK)], tid_v.at[b])
            gather_desc(b).start()

        def out_desc(c, b):
            base = base_w + c * CHUNK
            return pltpu.make_async_copy(wbuf.at[b], out_h.at[pl.ds(base, CHUNK)],
                                         osems[b])

        fire(0, 0)

        def pair_body(p, carry):
            c0 = 2 * p
            c1 = c0 + 1

            @pl.when(p > 0)
            def _():
                out_desc(c0 - 1, 1).wait()

            fire(c1, 1)
            gather_desc(0).wait()
            out_desc(c0, 0).start()

            @pl.when(p + 1 < NPAIR)
            def _():
                out_desc(c0, 0).wait()
                fire(c0 + 2, 0)

            gather_desc(1).wait()
            out_desc(c1, 1).start()
            return carry

        lax.fori_loop(0, NPAIR, pair_body, 0, unroll=False)
        out_desc(NCHUNK - 2, 0).wait()
        out_desc(NCHUNK - 1, 1).wait()

    return gather_kernel(tid, word_emb)


def _ln_tc(y, g, b):
    mu = jnp.mean(y, axis=-1, keepdims=True)
    d = y - mu
    var = jnp.mean(d * d, axis=-1, keepdims=True)
    return d * lax.rsqrt(var + EPS) * g + b


def _text_tc(wrows, pid, tt, pos_emb, w01, g, b):
    SUB = 1024
    ROWS = 8
    BM = ROWS * SUB
    NBLK = TOTAL // BM

    def body(w_ref, pid_ref, tt_ref, ptab_ref, w01_ref, g_ref, b_ref, o_ref):
        row0 = w01_ref[0:1, :]
        row1 = w01_ref[1:2, :]
        dims = (((0,), (0,)), ((), ()))
        for j in range(ROWS):
            pid_row = pid_ref[j:j + 1, :]
            ohT = (pid_row == lax.broadcasted_iota(jnp.int32, (MAXPOS, SUB), 0)
                   ).astype(jnp.float32)
            pos = lax.dot_general(ohT, ptab_ref[...], dims,
                                  preferred_element_type=jnp.float32)
            tt_row = tt_ref[j:j + 1, :].astype(jnp.float32)
            ttemb = lax.dot_general(tt_row, row1 - row0, dims,
                                    preferred_element_type=jnp.float32)
            sl = pl.ds(j * SUB, SUB)
            y = w_ref[sl, :] + pos + (ttemb + row0)
            o_ref[sl, :] = _ln_tc(y, g_ref[...], b_ref[...])

    row_spec = pl.BlockSpec((1, HID), lambda i: (0, 0))
    return pl.pallas_call(
        body,
        grid=(NBLK,),
        in_specs=[
            pl.BlockSpec((BM, HID), lambda i: (i, 0)),
            pl.BlockSpec((ROWS, SUB), lambda i: (i, 0)),
            pl.BlockSpec((ROWS, SUB), lambda i: (i, 0)),
            pl.BlockSpec((MAXPOS, HID), lambda i: (0, 0)),
            pl.BlockSpec((2, HID), lambda i: (0, 0)),
            row_spec, row_spec,
        ],
        out_specs=pl.BlockSpec((BM, HID), lambda i: (i, 0)),
        out_shape=jax.ShapeDtypeStruct((TOTAL, HID), jnp.float32),
    )(wrows, pid.reshape(TOTAL // SUB, SUB), tt.reshape(TOTAL // SUB, SUB),
      pos_emb, w01, g, b)


def _image_tc(image_flat, img_W, img_b, iln_g, iln_b, w1row, vln_g, vln_b):
    M = B * NB
    BM = 1024

    def body(x_ref, w_ref, b_ref, ig_ref, ib_ref, w1_ref, vg_ref, vb_ref, o_ref):
        y = jnp.dot(x_ref[...].astype(jnp.bfloat16),
                    w_ref[...].astype(jnp.bfloat16),
                    preferred_element_type=jnp.float32)
        y = y + b_ref[...]
        y = _ln_tc(y, ig_ref[...], ib_ref[...])
        y = y + w1_ref[...]
        o_ref[...] = _ln_tc(y, vg_ref[...], vb_ref[...])

    row_spec = pl.BlockSpec((1, HID), lambda i: (0, 0))
    return pl.pallas_call(
        body,
        grid=(M // BM,),
        in_specs=[
            pl.BlockSpec((BM, VDIM), lambda i: (i, 0)),
            pl.BlockSpec((VDIM, HID), lambda i: (0, 0)),
            row_spec, row_spec, row_spec, row_spec, row_spec, row_spec,
        ],
        out_specs=pl.BlockSpec((BM, HID), lambda i: (i, 0)),
        out_shape=jax.ShapeDtypeStruct((M, HID), jnp.float32),
    )(image_flat, img_W, img_b, iln_g, iln_b, w1row, vln_g, vln_b)


def kernel(token_ids, image_feat, token_type_ids, position_ids, word_emb,
           pos_emb, img_W, img_b, ln_g, ln_b, iln_g, iln_b, vln_g, vln_b):
    tid = token_ids.reshape(-1)
    wrows = _word_gather_sc(tid, word_emb)

    r = lambda a: a.reshape(1, HID)
    w01 = lax.slice(word_emb, (0, 0), (2, HID))
    w1row = lax.slice(word_emb, (1, 0), (2, HID))
    img_rows = image_feat.transpose(1, 0, 2).reshape(NB * B, VDIM)
    v = _image_tc(img_rows, img_W, r(img_b),
                  r(iln_g), r(iln_b), w1row, r(vln_g), r(vln_b))
    v = v.reshape(NB, B, HID).transpose(1, 0, 2)

    emb = _text_tc(wrows, position_ids.reshape(-1),
                   token_type_ids.reshape(-1), pos_emb, w01,
                   r(ln_g), r(ln_b)).reshape(B, S, HID)
    return (emb, v)


# speedup vs baseline: 1.0062x; 1.0062x over previous
import functools

import jax
import jax.numpy as jnp
from jax import lax
from jax.experimental import pallas as pl
from jax.experimental.pallas import tpu as pltpu
from jax.experimental.pallas import tpu_sc as plsc

VOCAB = 100000
HID = 128
MAXPOS = 512
VDIM = 2048
B = 1024
S = 200
NB = 36
EPS = 1e-12

NC = 2
NS = 16
NW = NC * NS
TOTAL = B * S
PER_W = TOTAL // NW
CHUNK = 128
NCHUNK = PER_W // CHUNK
NPAIR = NCHUNK // 2


def _word_gather_sc(tid, word_emb):
    mesh = plsc.VectorSubcoreMesh(core_axis_name="c", subcore_axis_name="s")

    @functools.partial(
        pl.kernel,
        out_type=jax.ShapeDtypeStruct((TOTAL, HID), jnp.float32),
        mesh=mesh,
        scratch_types=[
            pltpu.VMEM((2, CHUNK), jnp.int32),
            pltpu.VMEM((2, CHUNK, HID), jnp.float32),
            pltpu.SemaphoreType.DMA,
            pltpu.SemaphoreType.DMA,
            pltpu.SemaphoreType.DMA,
            pltpu.SemaphoreType.DMA,
        ],
    )
    def gather_kernel(tid_h, wtab_h, out_h, tid_v, wbuf, gs0, gs1, os0, os1):
        wid = lax.axis_index("s") * NC + lax.axis_index("c")
        base_w = wid * PER_W
        gsems = (gs0, gs1)
        osems = (os0, os1)

        def gather_desc(b):
            return pltpu.make_async_copy(wtab_h.at[tid_v.at[b]], wbuf.at[b],
                                         gsems[b])

        def fire(c, b):
            base = base_w + c * CHUNK
            pltpu.sync_copy(tid_h.at[pl.ds(base, CHUNK)], tid_v.at[b])
            gather_desc(b).start()

        def out_desc(c, b):
            base = base_w + c * CHUNK
            return pltpu.make_async_copy(wbuf.at[b], out_h.at[pl.ds(base, CHUNK)],
                                         osems[b])

        fire(0, 0)

        def pair_body(p, carry):
            c0 = 2 * p
            c1 = c0 + 1

            @pl.when(p > 0)
            def _():
                out_desc(c0 - 1, 1).wait()

            fire(c1, 1)
            gather_desc(0).wait()
            out_desc(c0, 0).start()

            @pl.when(p + 1 < NPAIR)
            def _():
                out_desc(c0, 0).wait()
                fire(c0 + 2, 0)

            gather_desc(1).wait()
            out_desc(c1, 1).start()
            return carry

        lax.fori_loop(0, NPAIR, pair_body, 0, unroll=False)
        out_desc(NCHUNK - 2, 0).wait()
        out_desc(NCHUNK - 1, 1).wait()

    return gather_kernel(tid, word_emb)


def _ln_tc(y, g, b):
    mu = jnp.mean(y, axis=-1, keepdims=True)
    d = y - mu
    var = jnp.mean(d * d, axis=-1, keepdims=True)
    return d * lax.rsqrt(var + EPS) * g + b


def _text_tc(wrows, pid, tt, pos_emb, w01, g, b):
    SUB = 1024
    ROWS = 8
    BM = ROWS * SUB
    NBLK = TOTAL // BM

    def body(w_ref, pid_ref, tt_ref, ptab_ref, w01_ref, g_ref, b_ref, o_ref):
        row0 = w01_ref[0:1, :]
        row1 = w01_ref[1:2, :]
        dims = (((0,), (0,)), ((), ()))
        for j in range(ROWS):
            pid_row = pid_ref[j:j + 1, :]
            ohT = (pid_row == lax.broadcasted_iota(jnp.int32, (MAXPOS, SUB), 0)
                   ).astype(jnp.float32)
            pos = lax.dot_general(ohT, ptab_ref[...], dims,
                                  preferred_element_type=jnp.float32)
            tt_row = tt_ref[j:j + 1, :].astype(jnp.float32)
            ttemb = lax.dot_general(tt_row, row1 - row0, dims,
                                    preferred_element_type=jnp.float32)
            sl = pl.ds(j * SUB, SUB)
            y = w_ref[sl, :] + pos + (ttemb + row0)
            o_ref[sl, :] = _ln_tc(y, g_ref[...], b_ref[...])

    row_spec = pl.BlockSpec((1, HID), lambda i: (0, 0))
    return pl.pallas_call(
        body,
        grid=(NBLK,),
        in_specs=[
            pl.BlockSpec((BM, HID), lambda i: (i, 0)),
            pl.BlockSpec((ROWS, SUB), lambda i: (i, 0)),
            pl.BlockSpec((ROWS, SUB), lambda i: (i, 0)),
            pl.BlockSpec((MAXPOS, HID), lambda i: (0, 0)),
            pl.BlockSpec((2, HID), lambda i: (0, 0)),
            row_spec, row_spec,
        ],
        out_specs=pl.BlockSpec((BM, HID), lambda i: (i, 0)),
        out_shape=jax.ShapeDtypeStruct((TOTAL, HID), jnp.float32),
    )(wrows, pid.reshape(TOTAL // SUB, SUB), tt.reshape(TOTAL // SUB, SUB),
      pos_emb, w01, g, b)


def _image_tc(image_flat, img_W, img_b, iln_g, iln_b, w1row, vln_g, vln_b):
    M = B * NB
    BM = 2048

    def body(x_ref, w_ref, b_ref, ig_ref, ib_ref, w1_ref, vg_ref, vb_ref, o_ref):
        y = jnp.dot(x_ref[...], w_ref[...], preferred_element_type=jnp.float32)
        y = y + b_ref[...]
        y = _ln_tc(y, ig_ref[...], ib_ref[...])
        y = y + w1_ref[...]
        o_ref[...] = _ln_tc(y, vg_ref[...], vb_ref[...])

    row_spec = pl.BlockSpec((1, HID), lambda i: (0, 0))
    return pl.pallas_call(
        body,
        grid=(M // BM,),
        in_specs=[
            pl.BlockSpec((BM, VDIM), lambda i: (i, 0)),
            pl.BlockSpec((VDIM, HID), lambda i: (0, 0)),
            row_spec, row_spec, row_spec, row_spec, row_spec, row_spec,
        ],
        out_specs=pl.BlockSpec((BM, HID), lambda i: (i, 0)),
        out_shape=jax.ShapeDtypeStruct((M, HID), jnp.float32),
    )(image_flat, img_W, img_b, iln_g, iln_b, w1row, vln_g, vln_b)


def kernel(token_ids, image_feat, token_type_ids, position_ids, word_emb,
           pos_emb, img_W, img_b, ln_g, ln_b, iln_g, iln_b, vln_g, vln_b):
    tid = token_ids.reshape(-1)
    wrows = _word_gather_sc(tid, word_emb)

    r = lambda a: a.reshape(1, HID)
    w01 = lax.slice(word_emb, (0, 0), (2, HID))
    w1row = lax.slice(word_emb, (1, 0), (2, HID))
    img_rows = image_feat.transpose(1, 0, 2).reshape(NB * B, VDIM)
    v = _image_tc(img_rows, img_W, r(img_b),
                  r(iln_g), r(iln_b), w1row, r(vln_g), r(vln_b))
    v = v.reshape(NB, B, HID).transpose(1, 0, 2)

    emb = _text_tc(wrows, position_ids.reshape(-1),
                   token_type_ids.reshape(-1), pos_emb, w01,
                   r(ln_g), r(ln_b)).reshape(B, S, HID)
    return (emb, v)


# speedup vs baseline: 1.0109x; 1.0047x over previous
import functools

import jax
import jax.numpy as jnp
from jax import lax
from jax.experimental import pallas as pl
from jax.experimental.pallas import tpu as pltpu
from jax.experimental.pallas import tpu_sc as plsc

VOCAB = 100000
HID = 128
MAXPOS = 512
VDIM = 2048
B = 1024
S = 200
NB = 36
EPS = 1e-12

NC = 2
NS = 16
NW = NC * NS
TOTAL = B * S
PER_W = TOTAL // NW
CHUNK = 128
NCHUNK = PER_W // CHUNK
NPAIR = NCHUNK // 2


def _word_gather_sc(tid, word_emb):
    mesh = plsc.VectorSubcoreMesh(core_axis_name="c", subcore_axis_name="s")

    @functools.partial(
        pl.kernel,
        out_type=jax.ShapeDtypeStruct((TOTAL, HID), jnp.float32),
        mesh=mesh,
        scratch_types=[
            pltpu.VMEM((2, CHUNK), jnp.int32),
            pltpu.VMEM((2, CHUNK, HID), jnp.float32),
            pltpu.SemaphoreType.DMA,
            pltpu.SemaphoreType.DMA,
            pltpu.SemaphoreType.DMA,
            pltpu.SemaphoreType.DMA,
        ],
    )
    def gather_kernel(tid_h, wtab_h, out_h, tid_v, wbuf, gs0, gs1, os0, os1):
        wid = lax.axis_index("s") * NC + lax.axis_index("c")
        base_w = wid * PER_W
        gsems = (gs0, gs1)
        osems = (os0, os1)

        def gather_desc(b):
            return pltpu.make_async_copy(wtab_h.at[tid_v.at[b]], wbuf.at[b],
                                         gsems[b])

        def fire(c, b):
            base = base_w + c * CHUNK
            pltpu.sync_copy(tid_h.at[pl.ds(base, CHUNK)], tid_v.at[b])
            gather_desc(b).start()

        def out_desc(c, b):
            base = base_w + c * CHUNK
            return pltpu.make_async_copy(wbuf.at[b], out_h.at[pl.ds(base, CHUNK)],
                                         osems[b])

        fire(0, 0)

        def pair_body(p, carry):
            c0 = 2 * p
            c1 = c0 + 1

            @pl.when(p > 0)
            def _():
                out_desc(c0 - 1, 1).wait()

            fire(c1, 1)
            gather_desc(0).wait()
            out_desc(c0, 0).start()

            @pl.when(p + 1 < NPAIR)
            def _():
                out_desc(c0, 0).wait()
                fire(c0 + 2, 0)

            gather_desc(1).wait()
            out_desc(c1, 1).start()
            return carry

        lax.fori_loop(0, NPAIR, pair_body, 0, unroll=False)
        out_desc(NCHUNK - 2, 0).wait()
        out_desc(NCHUNK - 1, 1).wait()

    return gather_kernel(tid, word_emb)


def _ln_tc(y, g, b):
    mu = jnp.mean(y, axis=-1, keepdims=True)
    d = y - mu
    var = jnp.mean(d * d, axis=-1, keepdims=True)
    return d * lax.rsqrt(var + EPS) * g + b


def _text_tc(wrows, pid, tt, pos_emb, w01, g, b):
    SUB = 1024
    ROWS = 8
    BM = ROWS * SUB
    NBLK = TOTAL // BM

    def body(w_ref, pid_ref, tt_ref, ptab_ref, w01_ref, g_ref, b_ref, o_ref):
        row0 = w01_ref[0:1, :]
        row1 = w01_ref[1:2, :]
        ptab_bf = ptab_ref[...].astype(jnp.bfloat16)
        dims = (((0,), (0,)), ((), ()))
        for j in range(ROWS):
            pid_row = pid_ref[j:j + 1, :]
            ohT = (pid_row == lax.broadcasted_iota(jnp.int32, (MAXPOS, SUB), 0)
                   ).astype(jnp.bfloat16)
            pos = lax.dot_general(ohT, ptab_bf, dims,
                                  preferred_element_type=jnp.float32)
            tt_row = tt_ref[j:j + 1, :].astype(jnp.float32)
            ttemb = lax.dot_general(tt_row, row1 - row0, dims,
                                    preferred_element_type=jnp.float32)
            sl = pl.ds(j * SUB, SUB)
            y = w_ref[sl, :] + pos + (ttemb + row0)
            o_ref[sl, :] = _ln_tc(y, g_ref[...], b_ref[...])

    row_spec = pl.BlockSpec((1, HID), lambda i: (0, 0))
    return pl.pallas_call(
        body,
        grid=(NBLK,),
        in_specs=[
            pl.BlockSpec((BM, HID), lambda i: (i, 0)),
            pl.BlockSpec((ROWS, SUB), lambda i: (i, 0)),
            pl.BlockSpec((ROWS, SUB), lambda i: (i, 0)),
            pl.BlockSpec((MAXPOS, HID), lambda i: (0, 0)),
            pl.BlockSpec((2, HID), lambda i: (0, 0)),
            row_spec, row_spec,
        ],
        out_specs=pl.BlockSpec((BM, HID), lambda i: (i, 0)),
        out_shape=jax.ShapeDtypeStruct((TOTAL, HID), jnp.float32),
    )(wrows, pid.reshape(TOTAL // SUB, SUB), tt.reshape(TOTAL // SUB, SUB),
      pos_emb, w01, g, b)


def _image_tc(image_flat, img_W, img_b, iln_g, iln_b, w1row, vln_g, vln_b):
    M = B * NB
    BM = 2048

    def body(x_ref, w_ref, b_ref, ig_ref, ib_ref, w1_ref, vg_ref, vb_ref, o_ref):
        y = jnp.dot(x_ref[...], w_ref[...], preferred_element_type=jnp.float32)
        y = y + b_ref[...]
        y = _ln_tc(y, ig_ref[...], ib_ref[...])
        y = y + w1_ref[...]
        o_ref[...] = _ln_tc(y, vg_ref[...], vb_ref[...])

    row_spec = pl.BlockSpec((1, HID), lambda i: (0, 0))
    return pl.pallas_call(
        body,
        grid=(M // BM,),
        in_specs=[
            pl.BlockSpec((BM, VDIM), lambda i: (i, 0)),
            pl.BlockSpec((VDIM, HID), lambda i: (0, 0)),
            row_spec, row_spec, row_spec, row_spec, row_spec, row_spec,
        ],
        out_specs=pl.BlockSpec((BM, HID), lambda i: (i, 0)),
        out_shape=jax.ShapeDtypeStruct((M, HID), jnp.float32),
    )(image_flat, img_W, img_b, iln_g, iln_b, w1row, vln_g, vln_b)


def kernel(token_ids, image_feat, token_type_ids, position_ids, word_emb,
           pos_emb, img_W, img_b, ln_g, ln_b, iln_g, iln_b, vln_g, vln_b):
    tid = token_ids.reshape(-1)
    wrows = _word_gather_sc(tid, word_emb)

    r = lambda a: a.reshape(1, HID)
    w01 = lax.slice(word_emb, (0, 0), (2, HID))
    w1row = lax.slice(word_emb, (1, 0), (2, HID))
    img_rows = image_feat.transpose(1, 0, 2).reshape(NB * B, VDIM)
    v = _image_tc(img_rows, img_W, r(img_b),
                  r(iln_g), r(iln_b), w1row, r(vln_g), r(vln_b))
    v = v.reshape(NB, B, HID).transpose(1, 0, 2)

    emb = _text_tc(wrows, position_ids.reshape(-1),
                   token_type_ids.reshape(-1), pos_emb, w01,
                   r(ln_g), r(ln_b)).reshape(B, S, HID)
    return (emb, v)
